# baseline (speedup 1.0000x reference)
import functools

import jax
import jax.numpy as jnp
from jax import lax
from jax.experimental import pallas as pl
from jax.experimental.pallas import tpu as pltpu
from jax.experimental.pallas import tpu_sc as plsc

NUM_FIELDS = 26
VOCAB = 100000
EMBED_DIM = 32
BATCH = 16384
TOTAL_DIM = NUM_FIELDS * EMBED_DIM
TOTAL_ROWS = BATCH * NUM_FIELDS

NUM_CORES = 2
NUM_SUBCORES = 16
NW = NUM_CORES * NUM_SUBCORES
ROWS_PER_W = TOTAL_ROWS // NW
CHUNK = 128
NCHUNK = ROWS_PER_W // CHUNK


def _make_gather():
    mesh = plsc.VectorSubcoreMesh(core_axis_name="c", subcore_axis_name="s")

    @functools.partial(
        pl.kernel,
        mesh=mesh,
        out_type=jax.ShapeDtypeStruct((TOTAL_ROWS, EMBED_DIM), jnp.float32),
        scratch_types=[
            pltpu.VMEM((ROWS_PER_W,), jnp.int32),
            pltpu.VMEM((2, CHUNK, EMBED_DIM), jnp.float32),
            pltpu.SemaphoreType.DMA,
            pltpu.SemaphoreType.DMA,
        ],
        compiler_params=pltpu.CompilerParams(use_tc_tiling_on_sc=False),
    )
    def gather_rows(idx_hbm, tab_hbm, out_hbm, idx_v, rows_v, sem0, sem1):
        wid = lax.axis_index("s") * NUM_CORES + lax.axis_index("c")
        base = wid * ROWS_PER_W
        pltpu.sync_copy(idx_hbm.at[pl.ds(base, ROWS_PER_W)], idx_v)

        sems = (sem0, sem1)

        def fire(i, buf):
            pltpu.async_copy(
                tab_hbm.at[idx_v.at[pl.ds(i * CHUNK, CHUNK)]],
                rows_v.at[buf],
                sems[buf],
            )

        def drain_store(i, buf):
            pltpu.make_async_copy(
                tab_hbm.at[idx_v.at[pl.ds(i * CHUNK, CHUNK)]],
                rows_v.at[buf],
                sems[buf],
            ).wait()
            pltpu.sync_copy(
                rows_v.at[buf], out_hbm.at[pl.ds(base + i * CHUNK, CHUNK)]
            )

        fire(0, 0)

        def body(j, _):
            i0 = j * 2
            fire(i0 + 1, 1)
            drain_store(i0, 0)
            fire(i0 + 2, 0)
            drain_store(i0 + 1, 1)
            return _

        lax.fori_loop(0, NCHUNK // 2 - 1, body, 0)
        i0 = NCHUNK - 2
        fire(i0 + 1, 1)
        drain_store(i0, 0)
        drain_store(i0 + 1, 1)

    return gather_rows


_gather_rows = _make_gather()


def _mm_body(a_ref, w_ref, b_ref, o_ref):
    o_ref[...] = (
        jnp.dot(a_ref[...], w_ref[...], preferred_element_type=jnp.float32)
        + b_ref[...]
    )


def _matmul(emb2d, W, b):
    BB = 1024
    return pl.pallas_call(
        _mm_body,
        grid=(BATCH // BB,),
        in_specs=[
            pl.BlockSpec((BB, TOTAL_DIM), lambda i: (i, 0)),
            pl.BlockSpec((TOTAL_DIM, EMBED_DIM), lambda i: (0, 0)),
            pl.BlockSpec((1, EMBED_DIM), lambda i: (0, 0)),
        ],
        out_specs=pl.BlockSpec((BB, EMBED_DIM), lambda i: (i, 0)),
        out_shape=jax.ShapeDtypeStruct((BATCH, EMBED_DIM), jnp.float32),
    )(emb2d, W, b.reshape(1, EMBED_DIM))


def kernel(x, tables, W, b):
    gidx = (
        x.astype(jnp.int32)
        + (jnp.arange(NUM_FIELDS, dtype=jnp.int32) * VOCAB)[None, :]
    ).reshape(-1)
    tab2 = tables.reshape(NUM_FIELDS * VOCAB, EMBED_DIM)
    emb = _gather_rows(gidx, tab2)
    return _matmul(emb.reshape(BATCH, TOTAL_DIM), W, b)


# speedup vs baseline: 1.6577x; 1.6577x over previous
import functools

import jax
import jax.numpy as jnp
from jax import lax
from jax.experimental import pallas as pl
from jax.experimental.pallas import tpu as pltpu
from jax.experimental.pallas import tpu_sc as plsc

NUM_FIELDS = 26
VOCAB = 100000
EMBED_DIM = 32
BATCH = 16384
TOTAL_DIM = NUM_FIELDS * EMBED_DIM
TOTAL_ROWS = BATCH * NUM_FIELDS

NUM_CORES = 2
NUM_SUBCORES = 16
NW = NUM_CORES * NUM_SUBCORES
ROWS_PER_W = TOTAL_ROWS // NW
CHUNK = 128
NCHUNK = ROWS_PER_W // CHUNK


VT = 2048
TGRID = (VOCAB + VT - 1) // VT


def _tr_body(tt_ref, o_ref):
    eye = jnp.eye(EMBED_DIM, dtype=jnp.float32)
    for f in range(NUM_FIELDS):
        for j in range(4):
            blk_j = tt_ref[f, :, j * (VT // 4):(j + 1) * (VT // 4)]
            t_j = lax.dot_general(
                blk_j, eye, (((0,), (0,)), ((), ())),
                preferred_element_type=jnp.float32,
            )
            o_ref[
                f * (VT // 4):(f + 1) * (VT // 4),
                j * EMBED_DIM:(j + 1) * EMBED_DIM,
            ] = t_j


def _build_flat_table(tt):
    return pl.pallas_call(
        _tr_body,
        grid=(TGRID,),
        in_specs=[
            pl.BlockSpec((NUM_FIELDS, EMBED_DIM, VT), lambda i: (0, 0, i)),
        ],
        out_specs=pl.BlockSpec(
            (NUM_FIELDS * VT // 4, 128), lambda i: (i, 0)
        ),
        out_shape=jax.ShapeDtypeStruct(
            (TGRID * NUM_FIELDS * VT // 4, 128), jnp.float32
        ),
    )(tt)


def _make_gather():
    mesh = plsc.VectorSubcoreMesh(core_axis_name="c", subcore_axis_name="s")

    @functools.partial(
        pl.kernel,
        mesh=mesh,
        out_type=jax.ShapeDtypeStruct((TOTAL_ROWS, EMBED_DIM), jnp.float32),
        scratch_types=[
            pltpu.VMEM((ROWS_PER_W,), jnp.int32),
            pltpu.VMEM((2, CHUNK, EMBED_DIM), jnp.float32),
            pltpu.SemaphoreType.DMA,
            pltpu.SemaphoreType.DMA,
        ],
        compiler_params=pltpu.CompilerParams(use_tc_tiling_on_sc=False),
    )
    def gather_rows(idx_hbm, tab_hbm, out_hbm, idx_v, rows_v, sem0, sem1):
        wid = lax.axis_index("s") * NUM_CORES + lax.axis_index("c")
        base = wid * ROWS_PER_W
        pltpu.sync_copy(idx_hbm.at[pl.ds(base, ROWS_PER_W)], idx_v)

        sems = (sem0, sem1)

        def fire(i, buf):
            pltpu.async_copy(
                tab_hbm.at[idx_v.at[pl.ds(i * CHUNK, CHUNK)]],
                rows_v.at[buf],
                sems[buf],
            )

        def drain_store(i, buf):
            pltpu.make_async_copy(
                tab_hbm.at[idx_v.at[pl.ds(i * CHUNK, CHUNK)]],
                rows_v.at[buf],
                sems[buf],
            ).wait()
            pltpu.sync_copy(
                rows_v.at[buf], out_hbm.at[pl.ds(base + i * CHUNK, CHUNK)]
            )

        fire(0, 0)

        def body(j, _):
            i0 = j * 2
            fire(i0 + 1, 1)
            drain_store(i0, 0)
            fire(i0 + 2, 0)
            drain_store(i0 + 1, 1)
            return _

        lax.fori_loop(0, NCHUNK // 2 - 1, body, 0)
        i0 = NCHUNK - 2
        fire(i0 + 1, 1)
        drain_store(i0, 0)
        drain_store(i0 + 1, 1)

    return gather_rows


_gather_rows = _make_gather()


def _mm_body(a_ref, w_ref, b_ref, o_ref):
    o_ref[...] = (
        jnp.dot(a_ref[...], w_ref[...], preferred_element_type=jnp.float32)
        + b_ref[...]
    )


def _matmul(emb2d, W, b):
    BB = 1024
    return pl.pallas_call(
        _mm_body,
        grid=(BATCH // BB,),
        in_specs=[
            pl.BlockSpec((BB, TOTAL_DIM), lambda i: (i, 0)),
            pl.BlockSpec((TOTAL_DIM, EMBED_DIM), lambda i: (0, 0)),
            pl.BlockSpec((1, EMBED_DIM), lambda i: (0, 0)),
        ],
        out_specs=pl.BlockSpec((BB, EMBED_DIM), lambda i: (i, 0)),
        out_shape=jax.ShapeDtypeStruct((BATCH, EMBED_DIM), jnp.float32),
    )(emb2d, W, b.reshape(1, EMBED_DIM))


def kernel(x, tables, W, b):
    tt = tables.astype(jnp.float32).transpose(0, 2, 1)
    flat = _build_flat_table(tt)
    v = x.astype(jnp.int32)
    f = jnp.arange(NUM_FIELDS, dtype=jnp.int32)[None, :]
    rr = v % VT
    gidx = (
        (v // VT) * (NUM_FIELDS * VT)
        + f * VT
        + 4 * (rr % (VT // 4))
        + rr // (VT // 4)
    ).reshape(-1)
    tab2 = flat.reshape(TGRID * NUM_FIELDS * VT, EMBED_DIM)
    emb = _gather_rows(gidx, tab2)
    return _matmul(emb.reshape(BATCH, TOTAL_DIM), W, b)


# speedup vs baseline: 1.6609x; 1.0019x over previous
import functools

import jax
import jax.numpy as jnp
from jax import lax
from jax.experimental import pallas as pl
from jax.experimental.pallas import tpu as pltpu
from jax.experimental.pallas import tpu_sc as plsc

NUM_FIELDS = 26
VOCAB = 100000
EMBED_DIM = 32
BATCH = 16384
TOTAL_DIM = NUM_FIELDS * EMBED_DIM
TOTAL_ROWS = BATCH * NUM_FIELDS

NUM_CORES = 2
NUM_SUBCORES = 16
NW = NUM_CORES * NUM_SUBCORES
ROWS_PER_W = TOTAL_ROWS // NW
CHUNK = 128
NCHUNK = ROWS_PER_W // CHUNK


VT = 2048
TGRID = (VOCAB + VT - 1) // VT


def _tr_body(tt_ref, o_ref):
    eye = jnp.eye(EMBED_DIM, dtype=jnp.float32)
    for f in range(NUM_FIELDS):
        t = lax.dot_general(
            tt_ref[f], eye, (((0,), (0,)), ((), ())),
            preferred_element_type=jnp.float32,
        )
        for j in range(4):
            o_ref[
                f * (VT // 4):(f + 1) * (VT // 4),
                j * EMBED_DIM:(j + 1) * EMBED_DIM,
            ] = t[j * (VT // 4):(j + 1) * (VT // 4)]


def _build_flat_table(tt):
    return pl.pallas_call(
        _tr_body,
        grid=(TGRID,),
        in_specs=[
            pl.BlockSpec((NUM_FIELDS, EMBED_DIM, VT), lambda i: (0, 0, i)),
        ],
        out_specs=pl.BlockSpec(
            (NUM_FIELDS * VT // 4, 128), lambda i: (i, 0)
        ),
        out_shape=jax.ShapeDtypeStruct(
            (TGRID * NUM_FIELDS * VT // 4, 128), jnp.float32
        ),
    )(tt)


def _make_gather():
    mesh = plsc.VectorSubcoreMesh(core_axis_name="c", subcore_axis_name="s")

    @functools.partial(
        pl.kernel,
        mesh=mesh,
        out_type=jax.ShapeDtypeStruct((TOTAL_ROWS, EMBED_DIM), jnp.float32),
        scratch_types=[
            pltpu.VMEM((ROWS_PER_W,), jnp.int32),
            pltpu.VMEM((2, CHUNK, EMBED_DIM), jnp.float32),
            pltpu.SemaphoreType.DMA,
            pltpu.SemaphoreType.DMA,
        ],
        compiler_params=pltpu.CompilerParams(use_tc_tiling_on_sc=False),
    )
    def gather_rows(idx_hbm, tab_hbm, out_hbm, idx_v, rows_v, sem0, sem1):
        wid = lax.axis_index("s") * NUM_CORES + lax.axis_index("c")
        base = wid * ROWS_PER_W
        pltpu.sync_copy(idx_hbm.at[pl.ds(base, ROWS_PER_W)], idx_v)

        sems = (sem0, sem1)

        def fire(i, buf):
            pltpu.async_copy(
                tab_hbm.at[idx_v.at[pl.ds(i * CHUNK, CHUNK)]],
                rows_v.at[buf],
                sems[buf],
            )

        def drain_store(i, buf):
            pltpu.make_async_copy(
                tab_hbm.at[idx_v.at[pl.ds(i * CHUNK, CHUNK)]],
                rows_v.at[buf],
                sems[buf],
            ).wait()
            pltpu.sync_copy(
                rows_v.at[buf], out_hbm.at[pl.ds(base + i * CHUNK, CHUNK)]
            )

        fire(0, 0)

        def body(j, _):
            i0 = j * 2
            fire(i0 + 1, 1)
            drain_store(i0, 0)
            fire(i0 + 2, 0)
            drain_store(i0 + 1, 1)
            return _

        lax.fori_loop(0, NCHUNK // 2 - 1, body, 0)
        i0 = NCHUNK - 2
        fire(i0 + 1, 1)
        drain_store(i0, 0)
        drain_store(i0 + 1, 1)

    return gather_rows


_gather_rows = _make_gather()


def _mm_body(a_ref, w_ref, b_ref, o_ref):
    o_ref[...] = (
        jnp.dot(a_ref[...], w_ref[...], preferred_element_type=jnp.float32)
        + b_ref[...]
    )


def _matmul(emb2d, W, b):
    BB = 1024
    return pl.pallas_call(
        _mm_body,
        grid=(BATCH // BB,),
        in_specs=[
            pl.BlockSpec((BB, TOTAL_DIM), lambda i: (i, 0)),
            pl.BlockSpec((TOTAL_DIM, EMBED_DIM), lambda i: (0, 0)),
            pl.BlockSpec((1, EMBED_DIM), lambda i: (0, 0)),
        ],
        out_specs=pl.BlockSpec((BB, EMBED_DIM), lambda i: (i, 0)),
        out_shape=jax.ShapeDtypeStruct((BATCH, EMBED_DIM), jnp.float32),
    )(emb2d, W, b.reshape(1, EMBED_DIM))


def kernel(x, tables, W, b):
    tt = tables.astype(jnp.float32).transpose(0, 2, 1)
    flat = _build_flat_table(tt)
    v = x.astype(jnp.int32)
    f = jnp.arange(NUM_FIELDS, dtype=jnp.int32)[None, :]
    rr = v % VT
    gidx = (
        (v // VT) * (NUM_FIELDS * VT)
        + f * VT
        + 4 * (rr % (VT // 4))
        + rr // (VT // 4)
    ).reshape(-1)
    tab2 = flat.reshape(TGRID * NUM_FIELDS * VT, EMBED_DIM)
    emb = _gather_rows(gidx, tab2)
    return _matmul(emb.reshape(BATCH, TOTAL_DIM), W, b)
